# baseline (speedup 1.0000x reference)
import functools

import jax
import jax.numpy as jnp
from jax import lax
from jax.experimental import pallas as pl
from jax.experimental.pallas import tpu as pltpu
from jax.experimental.pallas import tpu_sc as plsc

DOC_LEN = 16384
EMB = 64
NUM_TILES = 32
ROWS_PER_TILE = DOC_LEN // NUM_TILES
CHUNK = 128
NCHUNK = ROWS_PER_TILE // CHUNK


def _sc_stage(x0, x1, emb1, emb2):
    mesh = plsc.VectorSubcoreMesh(core_axis_name="c", subcore_axis_name="s")

    @functools.partial(
        pl.kernel,
        out_type=(
            jax.ShapeDtypeStruct((NUM_TILES, EMB), jnp.float32),
            jax.ShapeDtypeStruct((1, EMB), jnp.float32),
        ),
        mesh=mesh,
        compiler_params=pltpu.CompilerParams(use_tc_tiling_on_sc=False),
        scratch_types=[
            pltpu.VMEM((NCHUNK, CHUNK), jnp.int32),
            pltpu.VMEM((ROWS_PER_TILE, EMB), jnp.float32),
            pltpu.VMEM((EMB,), jnp.float32),
            pltpu.VMEM((1,), jnp.int32),
            pltpu.VMEM((1, EMB), jnp.float32),
            pltpu.SemaphoreType.DMA,
            pltpu.SemaphoreType.DMA,
        ],
    )
    def k(x0_hbm, x1_hbm, emb1_hbm, emb2_hbm, part_hbm, y2_hbm,
          idx_v, rows_v, acc_v, cp_v, y2_v, sem, sem2):
        wid = lax.axis_index("s") * 2 + lax.axis_index("c")
        base = wid * ROWS_PER_TILE

        for j in range(NCHUNK):
            pltpu.sync_copy(x0_hbm.at[pl.ds(base + j * CHUNK, CHUNK)],
                            idx_v.at[j])

        copies = []
        for j in range(NCHUNK):
            copies.append(pltpu.async_copy(
                emb1_hbm.at[idx_v.at[j]],
                rows_v.at[pl.ds(j * CHUNK, CHUNK)],
                sem))
        for c in copies:
            c.wait()

        @pl.when(wid == 0)
        def _():
            pltpu.sync_copy(x1_hbm, cp_v)
            pltpu.async_copy(emb2_hbm.at[cp_v], y2_v, sem2).wait()
            pltpu.sync_copy(y2_v, y2_hbm)

        zero = jnp.zeros((16,), jnp.float32)

        def body(i, accs):
            a0, a1, a2, a3 = accs
            a0 = a0 + rows_v[i, pl.ds(0, 16)]
            a1 = a1 + rows_v[i, pl.ds(16, 16)]
            a2 = a2 + rows_v[i, pl.ds(32, 16)]
            a3 = a3 + rows_v[i, pl.ds(48, 16)]
            return (a0, a1, a2, a3)

        a0, a1, a2, a3 = lax.fori_loop(0, ROWS_PER_TILE, body,
                                       (zero, zero, zero, zero))
        acc_v[pl.ds(0, 16)] = a0
        acc_v[pl.ds(16, 16)] = a1
        acc_v[pl.ds(32, 16)] = a2
        acc_v[pl.ds(48, 16)] = a3
        pltpu.sync_copy(acc_v, part_hbm.at[wid])

    return k(x0, x1, emb1, emb2)


def _tc_stage(partials, y2, x2, W1, b1, W2, b2):
    def body(p_ref, y2_ref, x2_ref, w1_ref, b1_ref, w2_ref, b2_ref, o_ref):
        s = jnp.sum(p_ref[...], axis=0, keepdims=True) * (1.0 / DOC_LEN)
        y1 = lax.dot_general(s, w1_ref[...], (((1,), (1,)), ((), ())),
                             preferred_element_type=jnp.float32) + b1_ref[...]
        y3 = jax.nn.sigmoid(
            lax.dot_general(x2_ref[...], w2_ref[...], (((1,), (1,)), ((), ())),
                            preferred_element_type=jnp.float32) + b2_ref[...])
        t = y2_ref[...] + y3
        o_ref[...] = jax.nn.sigmoid(jnp.sum(y1 * t, axis=1, keepdims=True))

    return pl.pallas_call(
        body,
        out_shape=jax.ShapeDtypeStruct((1, 1), jnp.float32),
    )(partials, y2, x2, W1, b1, W2, b2)


def kernel(x0, x1, x2, emb1, emb2, W1, b1, W2, b2):
    partials, y2 = _sc_stage(x0, x1, emb1, emb2)
    out = _tc_stage(partials, y2,
                    x2.reshape(1, EMB), W1, b1.reshape(1, EMB),
                    W2, b2.reshape(1, EMB))
    return out.reshape(())


# speedup vs baseline: 2.2228x; 2.2228x over previous
import functools

import jax
import jax.numpy as jnp
from jax import lax
from jax.experimental import pallas as pl
from jax.experimental.pallas import tpu as pltpu
from jax.experimental.pallas import tpu_sc as plsc

DOC_LEN = 16384
EMB = 64
NUM_TILES = 32
ROWS_PER_TILE = DOC_LEN // NUM_TILES
CHUNK = 32
NCHUNK = ROWS_PER_TILE // CHUNK


def _sc_stage(x0, x1, emb1_3d, emb2_3d):
    mesh = plsc.VectorSubcoreMesh(core_axis_name="c", subcore_axis_name="s")

    @functools.partial(
        pl.kernel,
        out_type=(
            jax.ShapeDtypeStruct((NUM_TILES, EMB, 16), jnp.float32),
            jax.ShapeDtypeStruct((1, 8, EMB), jnp.float32),
        ),
        mesh=mesh,
        compiler_params=pltpu.CompilerParams(needs_layout_passes=False),
        scratch_types=[
            pltpu.VMEM((ROWS_PER_TILE,), jnp.int32),
            pltpu.VMEM((ROWS_PER_TILE,), jnp.int32),
            pltpu.VMEM((2, CHUNK, 8, EMB), jnp.float32),
            pltpu.VMEM((EMB, 16), jnp.float32),
            pltpu.VMEM((16,), jnp.int32),
            pltpu.VMEM((1, 8, EMB), jnp.float32),
            pltpu.SemaphoreType.DMA,
            pltpu.SemaphoreType.DMA,
        ],
    )
    def k(x0_hbm, x1_hbm, emb1_hbm, emb2_hbm, part_hbm, y2_hbm,
          idx_v, tidx_v, tiles_v, acc_v, cp_v, y2_v, sem, sem2):
        wid = lax.axis_index("s") * 2 + lax.axis_index("c")
        base = wid * ROWS_PER_TILE

        pltpu.sync_copy(x0_hbm.at[pl.ds(base, ROWS_PER_TILE)], idx_v)

        for s in range(ROWS_PER_TILE // 16):
            tidx_v[pl.ds(s * 16, 16)] = (
                lax.shift_right_logical(idx_v[pl.ds(s * 16, 16)], 3))

        zero = jnp.zeros((16,), jnp.float32)
        for c in range(EMB):
            acc_v[c, :] = zero

        @pl.when(wid == 0)
        def _():
            x1v = x1_hbm.at[pl.ds(0, 1)]
            pltpu.sync_copy(x1v, cp_v.at[pl.ds(0, 1)])
            t1 = cp_v[pl.ds(0, 16)][0]
            pltpu.async_copy(emb2_hbm.at[t1], y2_v.at[0], sem2).wait()
            pltpu.sync_copy(y2_v, y2_hbm)

        lane = lax.iota(jnp.int32, 16)

        def fire(c, b):
            for g in range(CHUNK // 16):
                tv = tidx_v[pl.ds(c * CHUNK + g * 16, 16)]
                for j in range(16):
                    pltpu.async_copy(
                        emb1_hbm.at[tv[j]],
                        tiles_v.at[b].at[g * 16 + j], sem)

        def drain(b):
            for j in range(CHUNK):
                pltpu.make_async_copy(
                    emb1_hbm.at[0], tiles_v.at[b].at[j], sem).wait()

        def extract(c, b):
            for g in range(CHUNK // 16):
                iv = idx_v[pl.ds(c * CHUNK + g * 16, 16)]
                r = lax.bitwise_and(iv, jnp.full((16,), 7, jnp.int32))
                l = lane + g * 16
                for f in range(EMB):
                    v = plsc.load_gather(
                        tiles_v.at[b],
                        [l, r, jnp.full((16,), f, jnp.int32)])
                    plsc.addupdate(acc_v.at[f], v)

        fire(0, 0)
        fire(1, 1)

        def step(i, carry):
            for b in range(2):
                c = 2 * i + b
                drain(b)
                extract(c, b)

                @pl.when(c + 2 < NCHUNK)
                def _():
                    fire(c + 2, b)
            return carry

        lax.fori_loop(0, NCHUNK // 2, step, 0)

        pltpu.sync_copy(acc_v, part_hbm.at[wid])

    return k(x0, x1, emb1_3d, emb2_3d)


def _tc_stage(partials, y2tile, x1, x2, W1, b1, W2, b2):
    def body(x1_ref, p_ref, y2_ref, x2_ref, w1_ref, b1_ref, w2_ref, b2_ref,
             o_ref):
        s64 = jnp.sum(p_ref[...], axis=(0, 2)) * (1.0 / DOC_LEN)
        s = s64.reshape(1, EMB)
        y1 = lax.dot_general(s, w1_ref[...], (((1,), (1,)), ((), ())),
                             preferred_element_type=jnp.float32) + b1_ref[...]
        r = x1_ref[0] & 7
        y2 = y2_ref[0, pl.ds(r, 1), :]
        y3 = jax.nn.sigmoid(
            lax.dot_general(x2_ref[...], w2_ref[...], (((1,), (1,)), ((), ())),
                            preferred_element_type=jnp.float32) + b2_ref[...])
        t = y2 + y3
        o_ref[...] = jax.nn.sigmoid(jnp.sum(y1 * t, axis=1, keepdims=True))

    return pl.pallas_call(
        body,
        in_specs=[pl.BlockSpec(memory_space=pltpu.SMEM)]
        + [pl.BlockSpec()] * 7,
        out_shape=jax.ShapeDtypeStruct((1, 1), jnp.float32),
    )(x1, partials, y2tile, x2, W1, b1, W2, b2)


def kernel(x0, x1, x2, emb1, emb2, W1, b1, W2, b2):
    emb1_3d = emb1.reshape(emb1.shape[0] // 8, 8, EMB)
    emb2_3d = emb2.reshape(emb2.shape[0] // 8, 8, EMB)
    x1_tile = lax.shift_right_logical(x1, 3).astype(jnp.int32)
    partials, y2tile = _sc_stage(x0, x1_tile, emb1_3d, emb2_3d)
    out = _tc_stage(partials, y2tile, x1.astype(jnp.int32),
                    x2.reshape(1, EMB), W1, b1.reshape(1, EMB),
                    W2, b2.reshape(1, EMB))
    return out.reshape(())
